# Initial kernel scaffold; baseline (speedup 1.0000x reference)
import jax
import jax.numpy as jnp
from jax.experimental import pallas as pl


def kernel(X, A, T, W1, b1, Wg, bg, W00, b00, W10, b10, W01, b01, W11, b11):
    raise NotImplementedError("write your pallas kernel here")


# speedup vs baseline: 4299.3346x; 4299.3346x over previous
import jax
import jax.numpy as jnp
from jax import lax
from jax.experimental import pallas as pl

N = 2048
XD = 128
HD = 32
GD = 32
YREP = HD + GD + 1
BLK = 256
GRID = N // BLK

_DN = (((0,), (0,)), ((), ()))


def _pass1_body(a_ref, to_ref, x_ref, t_ref, w1_ref, b1_ref, wg_ref,
                stats_ref, phi_ref, xl_ref):
    stats_ref[...] = lax.dot_general(
        a_ref[...], to_ref[...], _DN, preferred_element_type=jnp.float32)
    phi = jax.nn.relu(
        jnp.dot(x_ref[...], w1_ref[...], preferred_element_type=jnp.float32)
        + b1_ref[...])
    phi_ref[...] = phi
    xl_ref[...] = jnp.dot(t_ref[...] * phi, wg_ref[...],
                          preferred_element_type=jnp.float32)


def _pass2_body(a_ref, xlf_ref, statsf_ref, stats_ref, phi_ref, xl_ref,
                bg_ref, w00_ref, b00_ref, w10_ref, b10_ref,
                w01_ref, b01_ref, w11_ref, b11_ref, out_ref):
    dinv_full = lax.rsqrt(statsf_ref[:, 1:2] + 1.0)
    bm = dinv_full * xlf_ref[...]
    cagg = lax.dot_general(a_ref[...], bm, _DN,
                           preferred_element_type=jnp.float32)
    stats_j = stats_ref[...]
    dinv_j = lax.rsqrt(stats_j[:, 1:2] + 1.0)
    xl_j = xl_ref[...]
    agg = dinv_j * (cagg + dinv_j * xl_j)
    rep_gnn = jax.nn.relu(agg + bg_ref[...])
    z = stats_j[:, 0:1] / stats_j[:, 1:2]
    rep = jnp.concatenate([phi_ref[...], rep_gnn, z], axis=1)
    y00 = jax.nn.relu(
        jnp.dot(rep, w00_ref[...], preferred_element_type=jnp.float32)
        + b00_ref[...])
    y10 = jax.nn.relu(
        jnp.dot(rep, w10_ref[...], preferred_element_type=jnp.float32)
        + b10_ref[...])
    y0 = jnp.dot(y00, w01_ref[...], preferred_element_type=jnp.float32) \
        + b01_ref[...]
    y1 = jnp.dot(y10, w11_ref[...], preferred_element_type=jnp.float32) \
        + b11_ref[...]
    out_ref[...] = jnp.concatenate([rep, y0, y1], axis=1)


def kernel(X, A, T, W1, b1, Wg, bg, W00, b00, W10, b10, W01, b01, W11, b11):
    f32 = jnp.float32
    t_col = T.reshape(N, 1).astype(f32)
    to = jnp.concatenate([t_col, jnp.ones((N, 1), f32)], axis=1)

    col_blk = pl.BlockSpec((N, BLK), lambda j: (0, j))
    row_blk = lambda w: pl.BlockSpec((BLK, w), lambda j: (j, 0))
    full = lambda a: pl.BlockSpec(a.shape, lambda j: (0,) * a.ndim)

    stats, phi, xl = pl.pallas_call(
        _pass1_body,
        grid=(GRID,),
        in_specs=[col_blk, full(to), row_blk(XD), row_blk(1),
                  full(W1), full(b1.reshape(1, HD)), full(Wg)],
        out_specs=[row_blk(2), row_blk(HD), row_blk(GD)],
        out_shape=[jax.ShapeDtypeStruct((N, 2), f32),
                   jax.ShapeDtypeStruct((N, HD), f32),
                   jax.ShapeDtypeStruct((N, GD), f32)],
    )(A, to, X, t_col, W1, b1.reshape(1, HD), Wg)

    out = pl.pallas_call(
        _pass2_body,
        grid=(GRID,),
        in_specs=[col_blk, full(xl), full(stats), row_blk(2),
                  row_blk(HD), row_blk(GD),
                  full(bg.reshape(1, GD)),
                  full(W00), full(b00.reshape(1, YREP)),
                  full(W10), full(b10.reshape(1, YREP)),
                  full(W01), full(b01.reshape(1, 1)),
                  full(W11), full(b11.reshape(1, 1))],
        out_specs=pl.BlockSpec((BLK, YREP + 2), lambda j: (j, 0)),
        out_shape=jax.ShapeDtypeStruct((N, YREP + 2), f32),
    )(A, xl, stats, stats, phi, xl,
      bg.reshape(1, GD), W00, b00.reshape(1, YREP),
      W10, b10.reshape(1, YREP), W01, b01.reshape(1, 1),
      W11, b11.reshape(1, 1))

    rep_post = out[:, :YREP]
    y0 = out[:, YREP]
    y1 = out[:, YREP + 1]
    return (y0, y1, rep_post)


# speedup vs baseline: 4943.0751x; 1.1497x over previous
import jax
import jax.numpy as jnp
from jax import lax
from jax.experimental import pallas as pl
from jax.experimental.pallas import tpu as pltpu

N = 2048
XD = 128
HD = 32
GD = 32
YREP = HD + GD + 1
BLK = 256
GRID = N // BLK

_DN = (((0,), (0,)), ((), ()))


def _body(a_ref, tf_ref, x_ref, t_ref, w1_ref, b1_ref, wg_ref, bg_ref,
          w00_ref, b00_ref, w10_ref, b10_ref, w01_ref, b01_ref,
          w11_ref, b11_ref,
          rep_ref, y0_ref, y1_ref,
          a_s, stats_s, phi_s, xl_s):
    j = pl.program_id(0)
    a_blk = a_ref[...]
    a_s[j] = a_blk
    to = jnp.concatenate(
        [tf_ref[...], jnp.ones((N, 1), jnp.float32)], axis=1)
    stats_s[pl.ds(j * BLK, BLK), :] = lax.dot_general(
        a_blk, to, _DN, preferred_element_type=jnp.float32)
    phi = jax.nn.relu(
        jnp.dot(x_ref[...], w1_ref[...], preferred_element_type=jnp.float32)
        + b1_ref[...])
    phi_s[pl.ds(j * BLK, BLK), :] = phi
    xl_s[pl.ds(j * BLK, BLK), :] = jnp.dot(
        t_ref[...] * phi, wg_ref[...], preferred_element_type=jnp.float32)

    @pl.when(j == GRID - 1)
    def _epilogue():
        stats = stats_s[...]
        dinv = lax.rsqrt(stats[:, 1:2] + 1.0)
        bm = dinv * xl_s[...]
        z_all = stats[:, 0:1] / stats[:, 1:2]
        for jb in range(GRID):
            lo = jb * BLK
            cagg = lax.dot_general(a_s[jb], bm, _DN,
                                   preferred_element_type=jnp.float32)
            dinv_j = dinv[lo:lo + BLK, :]
            agg = dinv_j * (cagg + dinv_j * xl_s[lo:lo + BLK, :])
            rep_gnn = jax.nn.relu(agg + bg_ref[...])
            rep = jnp.concatenate(
                [phi_s[lo:lo + BLK, :], rep_gnn, z_all[lo:lo + BLK, :]],
                axis=1)
            y00 = jax.nn.relu(
                jnp.dot(rep, w00_ref[...],
                        preferred_element_type=jnp.float32) + b00_ref[...])
            y10 = jax.nn.relu(
                jnp.dot(rep, w10_ref[...],
                        preferred_element_type=jnp.float32) + b10_ref[...])
            rep_ref[lo:lo + BLK, :] = rep
            y0_ref[lo:lo + BLK, :] = jnp.dot(
                y00, w01_ref[...],
                preferred_element_type=jnp.float32) + b01_ref[...]
            y1_ref[lo:lo + BLK, :] = jnp.dot(
                y10, w11_ref[...],
                preferred_element_type=jnp.float32) + b11_ref[...]


def kernel(X, A, T, W1, b1, Wg, bg, W00, b00, W10, b10, W01, b01, W11, b11):
    f32 = jnp.float32
    t_col = T.reshape(N, 1).astype(f32)

    col_blk = pl.BlockSpec((N, BLK), lambda j: (0, j))
    row_blk = lambda w: pl.BlockSpec((BLK, w), lambda j: (j, 0))
    full = lambda a: pl.BlockSpec(a.shape, lambda j: (0,) * a.ndim)
    const_out = lambda w: pl.BlockSpec((N, w), lambda j: (0, 0))

    rep_post, y0, y1 = pl.pallas_call(
        _body,
        grid=(GRID,),
        in_specs=[col_blk, full(t_col), row_blk(XD), row_blk(1),
                  full(W1), full(b1.reshape(1, HD)), full(Wg),
                  full(bg.reshape(1, GD)),
                  full(W00), full(b00.reshape(1, YREP)),
                  full(W10), full(b10.reshape(1, YREP)),
                  full(W01), full(b01.reshape(1, 1)),
                  full(W11), full(b11.reshape(1, 1))],
        out_specs=[const_out(YREP), const_out(1), const_out(1)],
        out_shape=[jax.ShapeDtypeStruct((N, YREP), f32),
                   jax.ShapeDtypeStruct((N, 1), f32),
                   jax.ShapeDtypeStruct((N, 1), f32)],
        scratch_shapes=[pltpu.VMEM((GRID, N, BLK), f32),
                        pltpu.VMEM((N, 2), f32),
                        pltpu.VMEM((N, HD), f32),
                        pltpu.VMEM((N, GD), f32)],
    )(A, t_col, X, t_col, W1, b1.reshape(1, HD), Wg,
      bg.reshape(1, GD), W00, b00.reshape(1, YREP),
      W10, b10.reshape(1, YREP), W01, b01.reshape(1, 1),
      W11, b11.reshape(1, 1))

    return (y0.reshape(-1), y1.reshape(-1), rep_post)


# speedup vs baseline: 5109.1821x; 1.0336x over previous
import jax
import jax.numpy as jnp
from jax import lax
from jax.experimental import pallas as pl
from jax.experimental.pallas import tpu as pltpu

N = 2048
XD = 128
HD = 32
GD = 32
YREP = HD + GD + 1
BLK = 256
GRID = N // BLK

_DN = (((0,), (0,)), ((), ()))
_F32 = jnp.float32
_BF16 = jnp.bfloat16


def _body(a_ref, x_ref, t_ref, w1_ref, b1_ref, wg_ref, bg_ref,
          w00_ref, b00_ref, w10_ref, b10_ref, w01_ref, b01_ref,
          w11_ref, b11_ref,
          rep_ref, y0_ref, y1_ref,
          a_s, stats_s, phi_s, xl_s):
    j = pl.program_id(0)
    ab = a_ref[...].astype(_BF16)
    a_s[j] = ab
    t_blk = t_ref[...]
    tob = jnp.concatenate(
        [t_blk, jnp.ones((BLK, 1), _F32)], axis=1).astype(_BF16)
    part = lax.dot_general(ab, tob, _DN,
                           preferred_element_type=_F32)

    @pl.when(j == 0)
    def _init():
        stats_s[...] = part

    @pl.when(j > 0)
    def _acc():
        stats_s[...] += part

    phi = jax.nn.relu(
        jnp.dot(x_ref[...], w1_ref[...], preferred_element_type=_F32)
        + b1_ref[...])
    phi_s[pl.ds(j * BLK, BLK), :] = phi
    xl_s[pl.ds(j * BLK, BLK), :] = jnp.dot(
        t_blk * phi, wg_ref[...], preferred_element_type=_F32)

    @pl.when(j == GRID - 1)
    def _epilogue():
        stats = stats_s[...]
        dinv = lax.rsqrt(stats[:, 1:2] + 1.0)
        z = stats[:, 0:1] / stats[:, 1:2]
        xl = xl_s[...]
        bm = dinv * xl
        bm_hi = bm.astype(_BF16)
        bm_lo = (bm - bm_hi.astype(_F32)).astype(_BF16)
        cagg = jnp.zeros((N, GD), _F32)
        for ib in range(GRID):
            lo = ib * BLK
            a_blk = a_s[ib]
            cagg = cagg + lax.dot_general(
                a_blk, bm_hi[lo:lo + BLK, :], _DN,
                preferred_element_type=_F32)
            cagg = cagg + lax.dot_general(
                a_blk, bm_lo[lo:lo + BLK, :], _DN,
                preferred_element_type=_F32)
        agg = dinv * (cagg + dinv * xl)
        rep_gnn = jax.nn.relu(agg + bg_ref[...])
        rep = jnp.concatenate([phi_s[...], rep_gnn, z], axis=1)
        y00 = jax.nn.relu(
            jnp.dot(rep, w00_ref[...], preferred_element_type=_F32)
            + b00_ref[...])
        y10 = jax.nn.relu(
            jnp.dot(rep, w10_ref[...], preferred_element_type=_F32)
            + b10_ref[...])
        rep_ref[...] = rep
        y0_ref[...] = jnp.dot(y00, w01_ref[...],
                              preferred_element_type=_F32) + b01_ref[...]
        y1_ref[...] = jnp.dot(y10, w11_ref[...],
                              preferred_element_type=_F32) + b11_ref[...]


def kernel(X, A, T, W1, b1, Wg, bg, W00, b00, W10, b10, W01, b01, W11, b11):
    t_col = T.reshape(N, 1).astype(_F32)

    row_blk = lambda w: pl.BlockSpec((BLK, w), lambda j: (j, 0))
    full = lambda a: pl.BlockSpec(a.shape, lambda j: (0,) * a.ndim)
    const_out = lambda w: pl.BlockSpec((N, w), lambda j: (0, 0))

    rep_post, y0, y1 = pl.pallas_call(
        _body,
        grid=(GRID,),
        in_specs=[row_blk(N), row_blk(XD), row_blk(1),
                  full(W1), full(b1.reshape(1, HD)), full(Wg),
                  full(bg.reshape(1, GD)),
                  full(W00), full(b00.reshape(1, YREP)),
                  full(W10), full(b10.reshape(1, YREP)),
                  full(W01), full(b01.reshape(1, 1)),
                  full(W11), full(b11.reshape(1, 1))],
        out_specs=[const_out(YREP), const_out(1), const_out(1)],
        out_shape=[jax.ShapeDtypeStruct((N, YREP), _F32),
                   jax.ShapeDtypeStruct((N, 1), _F32),
                   jax.ShapeDtypeStruct((N, 1), _F32)],
        scratch_shapes=[pltpu.VMEM((GRID, BLK, N), _BF16),
                        pltpu.VMEM((N, 2), _F32),
                        pltpu.VMEM((N, HD), _F32),
                        pltpu.VMEM((N, GD), _F32)],
    )(A, X, t_col, W1, b1.reshape(1, HD), Wg,
      bg.reshape(1, GD), W00, b00.reshape(1, YREP),
      W10, b10.reshape(1, YREP), W01, b01.reshape(1, 1),
      W11, b11.reshape(1, 1))

    return (y0.reshape(-1), y1.reshape(-1), rep_post)


# speedup vs baseline: 5434.3365x; 1.0636x over previous
import jax
import jax.numpy as jnp
from jax import lax
from jax.experimental import pallas as pl

N = 2048
XD = 128
HD = 32
GD = 32
YREP = HD + GD + 1

_DN = (((0,), (0,)), ((), ()))
_F32 = jnp.float32


def _body(a_ref, x_ref, t_ref, w1_ref, b1_ref, wg_ref, bg_ref,
          w00_ref, b00_ref, w10_ref, b10_ref, w01_ref, b01_ref,
          w11_ref, b11_ref,
          rep_ref, y0_ref, y1_ref):
    a = a_ref[...]
    t_col = t_ref[...]
    to = jnp.concatenate([t_col, jnp.ones((N, 1), _F32)], axis=1)
    stats = lax.dot_general(a, to, _DN,
                            preferred_element_type=_F32)
    dinv = lax.rsqrt(stats[:, 1:2] + 1.0)
    z = stats[:, 0:1] / stats[:, 1:2]
    phi = jax.nn.relu(
        jnp.dot(x_ref[...], w1_ref[...], preferred_element_type=_F32)
        + b1_ref[...])
    xl = jnp.dot(t_col * phi, wg_ref[...],
                 preferred_element_type=_F32)
    bm = dinv * xl
    cagg = lax.dot_general(a, bm, _DN,
                           preferred_element_type=_F32)
    agg = dinv * (cagg + dinv * xl)
    rep_gnn = jax.nn.relu(agg + bg_ref[...])
    rep = jnp.concatenate([phi, rep_gnn, z], axis=1)
    y00 = jax.nn.relu(
        jnp.dot(rep, w00_ref[...], preferred_element_type=_F32)
        + b00_ref[...])
    y10 = jax.nn.relu(
        jnp.dot(rep, w10_ref[...], preferred_element_type=_F32)
        + b10_ref[...])
    rep_ref[...] = rep
    y0_ref[...] = jnp.dot(y00, w01_ref[...],
                          preferred_element_type=_F32) + b01_ref[...]
    y1_ref[...] = jnp.dot(y10, w11_ref[...],
                          preferred_element_type=_F32) + b11_ref[...]


def kernel(X, A, T, W1, b1, Wg, bg, W00, b00, W10, b10, W01, b01, W11, b11):
    t_col = T.reshape(N, 1).astype(_F32)
    full = lambda a: pl.BlockSpec(a.shape, lambda: (0,) * a.ndim)

    args = (A, X, t_col, W1, b1.reshape(1, HD), Wg,
            bg.reshape(1, GD), W00, b00.reshape(1, YREP),
            W10, b10.reshape(1, YREP), W01, b01.reshape(1, 1),
            W11, b11.reshape(1, 1))

    rep_post, y0, y1 = pl.pallas_call(
        _body,
        in_specs=[full(a) for a in args],
        out_specs=[pl.BlockSpec((N, YREP), lambda: (0, 0)),
                   pl.BlockSpec((N, 1), lambda: (0, 0)),
                   pl.BlockSpec((N, 1), lambda: (0, 0))],
        out_shape=[jax.ShapeDtypeStruct((N, YREP), _F32),
                   jax.ShapeDtypeStruct((N, 1), _F32),
                   jax.ShapeDtypeStruct((N, 1), _F32)],
    )(*args)

    return (y0.reshape(-1), y1.reshape(-1), rep_post)
